# Initial kernel scaffold; baseline (speedup 1.0000x reference)
import jax
import jax.numpy as jnp
from jax.experimental import pallas as pl


def kernel(indices, table):
    raise NotImplementedError("write your pallas kernel here")


# speedup vs baseline: 3.9392x; 3.9392x over previous
import functools

import jax
import jax.numpy as jnp
from jax import lax
from jax.experimental import pallas as pl
from jax.experimental.pallas import tpu as pltpu
from jax.experimental.pallas import tpu_sc as plsc

_NC = 2
_NS = 16
_NW = _NC * _NS


def _embed_gather(idx_grouped, table, *, niter, chunk, embed_dim):
    n_rows = _NW * niter * chunk
    mesh = plsc.VectorSubcoreMesh(core_axis_name="c", subcore_axis_name="s")

    @functools.partial(
        pl.kernel,
        mesh=mesh,
        out_type=jax.ShapeDtypeStruct((n_rows, embed_dim), jnp.float32),
        scratch_types=[
            pltpu.VMEM((niter, chunk), jnp.int32),
            pltpu.VMEM((chunk, embed_dim), jnp.float32),
            pltpu.SemaphoreType.DMA,
        ],
        compiler_params=pltpu.CompilerParams(use_tc_tiling_on_sc=False),
    )
    def k(idx_hbm, table_hbm, out_hbm, idx_v, rows_v, sem):
        wid = lax.axis_index("s") * _NC + lax.axis_index("c")
        pltpu.sync_copy(idx_hbm.at[wid], idx_v)

        def step(i, carry):
            pltpu.async_copy(table_hbm.at[idx_v.at[i]], rows_v, sem).wait()
            base = (wid * niter + i) * chunk
            pltpu.sync_copy(rows_v, out_hbm.at[pl.ds(base, chunk)])
            return carry

        lax.fori_loop(0, niter, step, 0)

    return k(idx_grouped, table)


def kernel(indices, table):
    batch, hist = indices.shape
    vocab, embed_dim = table.shape
    n = batch * hist
    chunk = 3200
    niter = n // (_NW * chunk)
    idx_grouped = indices.reshape(_NW, niter, chunk)
    out = _embed_gather(idx_grouped, table, niter=niter, chunk=chunk,
                        embed_dim=embed_dim)
    return out.reshape(batch, hist, embed_dim)


# speedup vs baseline: 3.9587x; 1.0050x over previous
import functools

import jax
import jax.numpy as jnp
from jax import lax
from jax.experimental import pallas as pl
from jax.experimental.pallas import tpu as pltpu
from jax.experimental.pallas import tpu_sc as plsc

_NC = 2
_NS = 16
_NW = _NC * _NS


def _embed_gather(idx_grouped, table, *, niter, chunk, embed_dim):
    n_rows = _NW * niter * chunk
    mesh = plsc.VectorSubcoreMesh(core_axis_name="c", subcore_axis_name="s")

    @functools.partial(
        pl.kernel,
        mesh=mesh,
        out_type=jax.ShapeDtypeStruct((n_rows, embed_dim), jnp.float32),
        scratch_types=[
            pltpu.VMEM((niter, chunk), jnp.int32),
            pltpu.VMEM((chunk, embed_dim), jnp.float32),
            pltpu.VMEM((chunk, embed_dim), jnp.float32),
            pltpu.SemaphoreType.DMA,
            pltpu.SemaphoreType.DMA,
        ],
        compiler_params=pltpu.CompilerParams(use_tc_tiling_on_sc=False),
    )
    def k(idx_hbm, table_hbm, out_hbm, idx_v, rows0, rows1, sem0, sem1):
        wid = lax.axis_index("s") * _NC + lax.axis_index("c")
        pltpu.sync_copy(idx_hbm.at[wid], idx_v)

        def gather(i, buf, sem):
            return pltpu.async_copy(table_hbm.at[idx_v.at[i]], buf, sem)

        def wait_gather(i, buf, sem):
            pltpu.make_async_copy(table_hbm.at[idx_v.at[i]], buf, sem).wait()

        def scatter(i, buf):
            base = (wid * niter + i) * chunk
            pltpu.sync_copy(buf, out_hbm.at[pl.ds(base, chunk)])

        gather(0, rows0, sem0)

        def step2(j, carry):
            i0 = 2 * j
            gather(i0 + 1, rows1, sem1)
            wait_gather(i0, rows0, sem0)
            scatter(i0, rows0)

            @pl.when(j + 1 < niter // 2)
            def _():
                gather(i0 + 2, rows0, sem0)

            wait_gather(i0 + 1, rows1, sem1)
            scatter(i0 + 1, rows1)
            return carry

        lax.fori_loop(0, niter // 2, step2, 0)

    return k(idx_grouped, table)


def kernel(indices, table):
    batch, hist = indices.shape
    vocab, embed_dim = table.shape
    n = batch * hist
    chunk = 1600
    niter = n // (_NW * chunk)
    idx_grouped = indices.reshape(_NW, niter, chunk)
    out = _embed_gather(idx_grouped, table, niter=niter, chunk=chunk,
                        embed_dim=embed_dim)
    return out.reshape(batch, hist, embed_dim)


# speedup vs baseline: 5.9893x; 1.5129x over previous
import functools

import jax
import jax.numpy as jnp
from jax import lax
from jax.experimental import pallas as pl
from jax.experimental.pallas import tpu as pltpu
from jax.experimental.pallas import tpu_sc as plsc

_NC = 2
_NS = 16
_NW = _NC * _NS


def _embed_gather(idx_grouped, table, *, niter, chunk, embed_dim):
    n_rows = _NW * niter * chunk
    mesh = plsc.VectorSubcoreMesh(core_axis_name="c", subcore_axis_name="s")

    @functools.partial(
        pl.kernel,
        mesh=mesh,
        out_type=jax.ShapeDtypeStruct((n_rows, embed_dim), jnp.float32),
        scratch_types=[
            pltpu.VMEM((niter, chunk), jnp.int32),
            pltpu.VMEM_SHARED(table.shape, jnp.float32),
            pltpu.VMEM((chunk, embed_dim), jnp.float32),
            pltpu.VMEM((chunk, embed_dim), jnp.float32),
            pltpu.SemaphoreType.DMA,
            pltpu.SemaphoreType.DMA,
        ],
        compiler_params=pltpu.CompilerParams(use_tc_tiling_on_sc=False),
    )
    def k(idx_hbm, table_hbm, out_hbm, idx_v, table_v, rows0, rows1, sem0, sem1):
        sid = lax.axis_index("s")
        wid = sid * _NC + lax.axis_index("c")

        @pl.when(sid == 0)
        def _():
            pltpu.sync_copy(table_hbm, table_v)

        pltpu.sync_copy(idx_hbm.at[wid], idx_v)
        plsc.subcore_barrier()

        def gather(i, buf, sem):
            return pltpu.async_copy(table_v.at[idx_v.at[i]], buf, sem)

        def wait_gather(i, buf, sem):
            pltpu.make_async_copy(table_v.at[idx_v.at[i]], buf, sem).wait()

        def scatter(i, buf):
            base = (wid * niter + i) * chunk
            pltpu.sync_copy(buf, out_hbm.at[pl.ds(base, chunk)])

        gather(0, rows0, sem0)

        def step2(j, carry):
            i0 = 2 * j
            gather(i0 + 1, rows1, sem1)
            wait_gather(i0, rows0, sem0)
            scatter(i0, rows0)

            @pl.when(j + 1 < niter // 2)
            def _():
                gather(i0 + 2, rows0, sem0)

            wait_gather(i0 + 1, rows1, sem1)
            scatter(i0 + 1, rows1)
            return carry

        lax.fori_loop(0, niter // 2, step2, 0)

    return k(idx_grouped, table)


def kernel(indices, table):
    batch, hist = indices.shape
    vocab, embed_dim = table.shape
    n = batch * hist
    chunk = 1600
    niter = n // (_NW * chunk)
    idx_grouped = indices.reshape(_NW, niter, chunk)
    out = _embed_gather(idx_grouped, table, niter=niter, chunk=chunk,
                        embed_dim=embed_dim)
    return out.reshape(batch, hist, embed_dim)
